# Initial kernel scaffold; baseline (speedup 1.0000x reference)
import jax
import jax.numpy as jnp
from jax.experimental import pallas as pl


def kernel(probas, h_d, mask, batch_vocab, emb_table):
    raise NotImplementedError("write your pallas kernel here")


# speedup vs baseline: 7.7166x; 7.7166x over previous
import functools

import jax
import jax.numpy as jnp
from jax import lax
from jax.experimental import pallas as pl
from jax.experimental.pallas import tpu as pltpu
from jax.experimental.pallas import tpu_sc as plsc

_TOP_K = 16
_N_ITER = 4
_EARLY = 2
_W = 512
_R = 8
_L = 32


def _pass1_body(p_ref, g_ref, x_ref, smp_ref):
    R, V = p_ref.shape
    K = _TOP_K
    W = _W
    nfull = V // W
    tailw = V - nfull * W
    C = nfull + (1 if tailw else 0)
    CP = 256

    lane_cp = lax.broadcasted_iota(jnp.int32, (R, CP), 1)
    M1 = jnp.full((R, CP), -2.0, dtype=jnp.float32)
    rs = jnp.zeros((R, 1), dtype=jnp.float32)
    for c in range(nfull):
        ch = p_ref[:, c * W:(c + 1) * W]
        rs = rs + jnp.sum(ch, axis=1, keepdims=True)
        mx = jnp.max(ch, axis=1, keepdims=True)
        M1 = jnp.where(lane_cp == c, mx, M1)
    if tailw:
        ch = p_ref[:, nfull * W:]
        rs = rs + jnp.sum(ch, axis=1, keepdims=True)
        mx = jnp.max(ch, axis=1, keepdims=True)
        M1 = jnp.where(lane_cp == nfull, mx, M1)

    recip = 0.5 / (0.5 * rs)
    for c in range(nfull):
        sl = slice(c * W, (c + 1) * W)
        x_ref[:, sl] = p_ref[:, sl] * recip
    if tailw:
        x_ref[:, nfull * W:] = p_ref[:, nfull * W:] * recip

    BIG = jnp.int32(1 << 30)
    lane_w = lax.broadcasted_iota(jnp.int32, (1, W), 1)
    lane_t = lax.broadcasted_iota(jnp.int32, (1, tailw), 1)
    k_iota = lax.broadcasted_iota(jnp.int32, (1, K), 1)
    it_iota = lax.broadcasted_iota(jnp.int32, (1, _N_ITER), 1)
    tbase = nfull * W
    for r in range(R):
        M1r = M1[r:r + 1, :]
        lane_r = lane_cp[0:1, :]
        vals = jnp.zeros((1, K), dtype=jnp.float32)
        idxs = jnp.zeros((1, K), dtype=jnp.int32)
        for k in range(K):
            m = jnp.max(M1r)
            c = jnp.min(jnp.where(M1r == m, lane_r, BIG))
            cw = jnp.minimum(c, nfull - 1)
            start = pl.multiple_of(cw * W, W)
            win = p_ref[pl.ds(r, 1), pl.ds(start, W)]
            posw = jnp.min(jnp.where(win == m, lane_w, BIG)) + cw * W
            tl = p_ref[pl.ds(r, 1), pl.ds(tbase, tailw)]
            post = jnp.min(jnp.where(tl == m, lane_t, BIG)) + tbase
            is_tail = c == (C - 1)
            pos = jnp.where(is_tail, post, posw)
            vals = jnp.where(k_iota == k, m, vals)
            idxs = jnp.where(k_iota == k, pos, idxs)
            win2 = jnp.where(lane_w + cw * W == pos, -1.0, win)
            p_ref[pl.ds(r, 1), pl.ds(start, W)] = win2
            tl2 = jnp.where(lane_t + tbase == pos, -1.0, tl)
            p_ref[pl.ds(r, 1), pl.ds(tbase, tailw)] = tl2
            newm = jnp.where(is_tail, jnp.max(tl2), jnp.max(win2))
            M1r = jnp.where(lane_r == c, newm, M1r)
        lgr = jnp.log(vals)
        is_last = (pl.program_id(0) * R + r) % _L == (_L - 1)
        acc = jnp.zeros((1, _N_ITER), dtype=jnp.int32)
        for it in range(_N_ITER):
            s = lgr + g_ref[it, pl.ds(r, 1), :]
            m2 = jnp.max(s)
            ch = jnp.min(jnp.where(s == m2, k_iota, BIG))
            ch = jnp.where(is_last, 0, ch)
            samp = jnp.min(jnp.where(k_iota == ch, idxs, BIG))
            acc = jnp.where(it_iota == it, samp, acc)
        smp_ref[pl.ds(r, 1), :] = acc


def _pass1(p2, gum3):
    N, V = p2.shape
    grid = N // _R
    return pl.pallas_call(
        _pass1_body,
        grid=(grid,),
        in_specs=[
            pl.BlockSpec((_R, V), lambda i: (i, 0)),
            pl.BlockSpec((_N_ITER, _R, _TOP_K), lambda i: (0, i, 0)),
        ],
        out_specs=[
            pl.BlockSpec((_R, V), lambda i: (i, 0)),
            pl.BlockSpec((_R, _N_ITER), lambda i: (i, 0)),
        ],
        out_shape=[
            jax.ShapeDtypeStruct((N, V), jnp.float32),
            jax.ShapeDtypeStruct((N, _N_ITER), jnp.int32),
        ],
    )(p2, gum3)


def _sc_gather(samples1, batch_vocab, emb_table):
    n_tasks = samples1.shape[0]
    D = emb_table.shape[1]
    info = plsc.get_sparse_core_info()
    NC, NS = info.num_cores, info.num_subcores
    NW = NC * NS
    per_w = n_tasks // NW
    mesh = plsc.VectorSubcoreMesh(core_axis_name="c", subcore_axis_name="s")

    @functools.partial(
        pl.kernel,
        mesh=mesh,
        out_type=jax.ShapeDtypeStruct((n_tasks, D), jnp.float32),
        scratch_types=[
            pltpu.VMEM((per_w,), jnp.int32),
            pltpu.VMEM((per_w,), jnp.int32),
            pltpu.VMEM((per_w, D), jnp.float32),
            pltpu.SemaphoreType.DMA,
        ],
    )
    def body(s_hbm, bv_hbm, emb_hbm, out_hbm, smp_v, wi_v, rows_v, sem):
        wid = lax.axis_index("s") * NC + lax.axis_index("c")
        base = wid * per_w
        pltpu.sync_copy(s_hbm.at[pl.ds(base, per_w)], smp_v)
        pltpu.async_copy(bv_hbm.at[smp_v], wi_v, sem).wait()
        pltpu.async_copy(emb_hbm.at[wi_v], rows_v, sem).wait()
        pltpu.sync_copy(rows_v, out_hbm.at[pl.ds(base, per_w)])

    return body(samples1, batch_vocab, emb_table)


def _det_body(e_ref, h_ref, out_ref):
    NM = e_ref.shape[0]
    Lm = e_ref.shape[1]
    B = h_ref.shape[0]
    row_i = lax.broadcasted_iota(jnp.int32, (Lm, Lm), 0)
    col_i = lax.broadcasted_iota(jnp.int32, (Lm, Lm), 1)
    b_iota = lax.broadcasted_iota(jnp.int32, (1, B), 1)
    dets = [None] * NM
    for m in range(NM):
        b = m % B
        G = (e_ref[m] + h_ref[b]) / 10.0
        A0 = lax.dot_general(G, G, (((1,), (1,)), ((), ())),
                             preferred_element_type=jnp.float32)

        def step(j, carry):
            A, d = carry
            rsel = row_i == j
            csel = col_i == j
            prow = jnp.sum(jnp.where(rsel, A, 0.0), axis=0, keepdims=True)
            pv = jnp.sum(jnp.where(jnp.logical_and(rsel, csel), A, 0.0))
            fcol = jnp.sum(jnp.where(csel, A, 0.0), axis=1, keepdims=True) / pv
            upd_mask = row_i > j
            A = A - jnp.where(upd_mask, fcol * prow, 0.0)
            return A, d * pv

        _, det_m = lax.fori_loop(0, Lm, step,
                                 (A0, jnp.float32(1.0)), unroll=False)
        dets[m] = det_m
    S = [dets[i * B:(i + 1) * B] for i in range(_N_ITER)]

    def rowvec(ds):
        v = jnp.zeros((1, B), jnp.float32)
        for b in range(B):
            v = jnp.where(b_iota == b, ds[b], v)
        return v

    Srows = [rowvec(ds) for ds in S]
    ms = Srows[0]
    cnt = jnp.int32(0)
    done = jnp.bool_(False)
    for it in range(1, _N_ITER):
        s = Srows[it]
        mvec = ms < s
        any_m = jnp.any(mvec)
        cnt_new = jnp.where(any_m, jnp.int32(0), cnt + 1)
        cnt = jnp.where(done, cnt, cnt_new)
        hit = cnt >= _EARLY
        upd = jnp.logical_and(jnp.logical_not(done), jnp.logical_not(hit))
        done = jnp.logical_or(done, hit)
        ms = jnp.where(jnp.logical_and(upd, mvec), s, ms)
    out_ref[...] = ms


def _det(embs3, h_d):
    B = h_d.shape[0]
    return pl.pallas_call(
        _det_body,
        out_shape=jax.ShapeDtypeStruct((1, B), jnp.float32),
    )(embs3, h_d)


def kernel(probas, h_d, mask, batch_vocab, emb_table):
    B, L, V = probas.shape
    N = B * L
    p2 = probas.reshape(N, V)
    key = jax.random.key(42)
    gum3 = jnp.stack(
        [jax.random.gumbel(jax.random.fold_in(key, it), (N, _TOP_K),
                           jnp.float32) for it in range(_N_ITER)], axis=0)
    x2, samples = _pass1(p2, gum3)

    embs = _sc_gather(samples.reshape(-1), batch_vocab, emb_table)
    embs3 = embs.reshape(B, L, _N_ITER, 128).transpose(2, 0, 1, 3)
    ms = _det(embs3.reshape(_N_ITER * B, L, 128), h_d)
    return (x2.reshape(B, L, V), ms.reshape(B))


# speedup vs baseline: 53.9618x; 6.9930x over previous
import functools

import jax
import jax.numpy as jnp
from jax import lax
from jax.experimental import pallas as pl
from jax.experimental.pallas import tpu as pltpu
from jax.experimental.pallas import tpu_sc as plsc

_TOP_K = 16
_N_ITER = 4
_EARLY = 2
_W = 512
_R = 8
_L = 32


def _pass1_body(p_ref, g_ref, x_ref, smp_ref, win_ref):
    R, V = p_ref.shape
    K = _TOP_K
    W = _W
    nfull = V // W
    tailw = V - nfull * W
    C = nfull + (1 if tailw else 0)
    CP = 256

    lane_cp = lax.broadcasted_iota(jnp.int32, (R, CP), 1)
    M1 = jnp.full((R, CP), -2.0, dtype=jnp.float32)
    rs = jnp.zeros((R, 1), dtype=jnp.float32)
    for c in range(nfull):
        ch = p_ref[:, c * W:(c + 1) * W]
        rs = rs + jnp.sum(ch, axis=1, keepdims=True)
        mx = jnp.max(ch, axis=1, keepdims=True)
        M1 = jnp.where(lane_cp == c, mx, M1)
    tl_all = p_ref[:, nfull * W:]
    rs = rs + jnp.sum(tl_all, axis=1, keepdims=True)
    mx = jnp.max(tl_all, axis=1, keepdims=True)
    M1 = jnp.where(lane_cp == nfull, mx, M1)

    recip = 0.5 / (0.5 * rs)
    for c in range(nfull):
        sl = slice(c * W, (c + 1) * W)
        x_ref[:, sl] = p_ref[:, sl] * recip
    x_ref[:, nfull * W:] = tl_all * recip

    BIG = jnp.int32(1 << 30)
    lane_w = lax.broadcasted_iota(jnp.int32, (R, W), 1)
    lane_t = lax.broadcasted_iota(jnp.int32, (R, tailw), 1)
    k_iota = lax.broadcasted_iota(jnp.int32, (R, K), 1)
    it_iota = lax.broadcasted_iota(jnp.int32, (R, _N_ITER), 1)
    tbase = nfull * W
    glpos_t = lane_t + tbase
    vals = jnp.zeros((R, K), dtype=jnp.float32)
    idxs = jnp.zeros((R, K), dtype=jnp.int32)
    for k in range(K):
        m8 = jnp.max(M1, axis=1, keepdims=True)
        c8 = jnp.min(jnp.where(M1 == m8, lane_cp, BIG), axis=1,
                     keepdims=True)
        cw8 = jnp.minimum(c8, nfull - 1)
        for r in range(R):
            start = pl.multiple_of(cw8[r, 0] * W, W)
            win_ref[pl.ds(r, 1), :] = p_ref[pl.ds(r, 1), pl.ds(start, W)]
        g = win_ref[...]
        glpos = lane_w + cw8 * W
        is_tail = c8 == (C - 1)
        p8w = jnp.min(jnp.where(g == m8, glpos, BIG), axis=1, keepdims=True)
        p8t = jnp.min(jnp.where(tl_all == m8, glpos_t, BIG), axis=1,
                      keepdims=True)
        p8 = jnp.where(is_tail, p8t, p8w)
        vals = jnp.where(k_iota == k, m8, vals)
        idxs = jnp.where(k_iota == k, p8, idxs)
        if k < K - 1:
            keepw = jnp.logical_or(g < m8,
                                   jnp.logical_and(g == m8, glpos > p8))
            nmw = jnp.max(jnp.where(keepw, g, -2.0), axis=1, keepdims=True)
            keept = jnp.logical_or(tl_all < m8,
                                   jnp.logical_and(tl_all == m8,
                                                   glpos_t > p8))
            nmt = jnp.max(jnp.where(keept, tl_all, -2.0), axis=1,
                          keepdims=True)
            newm = jnp.where(is_tail, nmt, nmw)
            M1 = jnp.where(lane_cp == c8, newm, M1)

    lgr = jnp.log(vals)
    rowid = pl.program_id(0) * R + lax.broadcasted_iota(
        jnp.int32, (R, 1), 0)
    is_last = rowid % _L == (_L - 1)
    acc = jnp.zeros((R, _N_ITER), dtype=jnp.int32)
    for it in range(_N_ITER):
        s = lgr + g_ref[it]
        m2 = jnp.max(s, axis=1, keepdims=True)
        ch = jnp.min(jnp.where(s == m2, k_iota, BIG), axis=1, keepdims=True)
        ch = jnp.where(is_last, 0, ch)
        samp = jnp.min(jnp.where(k_iota == ch, idxs, BIG), axis=1,
                       keepdims=True)
        acc = jnp.where(it_iota == it, samp, acc)
    smp_ref[...] = acc


def _pass1(p2, gum3):
    N, V = p2.shape
    grid = N // _R
    return pl.pallas_call(
        _pass1_body,
        grid=(grid,),
        in_specs=[
            pl.BlockSpec((_R, V), lambda i: (i, 0)),
            pl.BlockSpec((_N_ITER, _R, _TOP_K), lambda i: (0, i, 0)),
        ],
        out_specs=[
            pl.BlockSpec((_R, V), lambda i: (i, 0)),
            pl.BlockSpec((_R, _N_ITER), lambda i: (i, 0)),
        ],
        out_shape=[
            jax.ShapeDtypeStruct((N, V), jnp.float32),
            jax.ShapeDtypeStruct((N, _N_ITER), jnp.int32),
        ],
        scratch_shapes=[pltpu.VMEM((_R, _W), jnp.float32)],
    )(p2, gum3)


def _sc_gather(samples1, batch_vocab, emb_table):
    n_tasks = samples1.shape[0]
    D = emb_table.shape[1]
    info = plsc.get_sparse_core_info()
    NC, NS = info.num_cores, info.num_subcores
    NW = NC * NS
    per_w = n_tasks // NW
    mesh = plsc.VectorSubcoreMesh(core_axis_name="c", subcore_axis_name="s")

    @functools.partial(
        pl.kernel,
        mesh=mesh,
        out_type=jax.ShapeDtypeStruct((n_tasks, D), jnp.float32),
        scratch_types=[
            pltpu.VMEM((per_w,), jnp.int32),
            pltpu.VMEM((per_w,), jnp.int32),
            pltpu.VMEM((per_w, D), jnp.float32),
            pltpu.SemaphoreType.DMA,
        ],
    )
    def body(s_hbm, bv_hbm, emb_hbm, out_hbm, smp_v, wi_v, rows_v, sem):
        wid = lax.axis_index("s") * NC + lax.axis_index("c")
        base = wid * per_w
        pltpu.sync_copy(s_hbm.at[pl.ds(base, per_w)], smp_v)
        pltpu.async_copy(bv_hbm.at[smp_v], wi_v, sem).wait()
        pltpu.async_copy(emb_hbm.at[wi_v], rows_v, sem).wait()
        pltpu.sync_copy(rows_v, out_hbm.at[pl.ds(base, per_w)])

    return body(samples1, batch_vocab, emb_table)


def _det_body(e_ref, h_ref, out_ref):
    NM = e_ref.shape[0]
    Lm = e_ref.shape[1]
    B = h_ref.shape[0]
    row_i = lax.broadcasted_iota(jnp.int32, (Lm, Lm), 0)
    col_i = lax.broadcasted_iota(jnp.int32, (Lm, Lm), 1)
    b_iota = lax.broadcasted_iota(jnp.int32, (1, B), 1)
    dets = [None] * NM
    for m in range(NM):
        b = m % B
        G = (e_ref[m] + h_ref[b]) / 10.0
        A0 = lax.dot_general(G, G, (((1,), (1,)), ((), ())),
                             preferred_element_type=jnp.float32)

        def step(j, carry):
            A, d = carry
            rsel = row_i == j
            csel = col_i == j
            prow = jnp.sum(jnp.where(rsel, A, 0.0), axis=0, keepdims=True)
            pv = jnp.sum(jnp.where(jnp.logical_and(rsel, csel), A, 0.0))
            fcol = jnp.sum(jnp.where(csel, A, 0.0), axis=1, keepdims=True) / pv
            upd_mask = row_i > j
            A = A - jnp.where(upd_mask, fcol * prow, 0.0)
            return A, d * pv

        _, det_m = lax.fori_loop(0, Lm, step,
                                 (A0, jnp.float32(1.0)), unroll=False)
        dets[m] = det_m
    S = [dets[i * B:(i + 1) * B] for i in range(_N_ITER)]

    def rowvec(ds):
        v = jnp.zeros((1, B), jnp.float32)
        for b in range(B):
            v = jnp.where(b_iota == b, ds[b], v)
        return v

    Srows = [rowvec(ds) for ds in S]
    ms = Srows[0]
    cnt = jnp.int32(0)
    done = jnp.bool_(False)
    for it in range(1, _N_ITER):
        s = Srows[it]
        mvec = ms < s
        any_m = jnp.any(mvec)
        cnt_new = jnp.where(any_m, jnp.int32(0), cnt + 1)
        cnt = jnp.where(done, cnt, cnt_new)
        hit = cnt >= _EARLY
        upd = jnp.logical_and(jnp.logical_not(done), jnp.logical_not(hit))
        done = jnp.logical_or(done, hit)
        ms = jnp.where(jnp.logical_and(upd, mvec), s, ms)
    out_ref[...] = ms


def _det(embs3, h_d):
    B = h_d.shape[0]
    return pl.pallas_call(
        _det_body,
        out_shape=jax.ShapeDtypeStruct((1, B), jnp.float32),
    )(embs3, h_d)


def kernel(probas, h_d, mask, batch_vocab, emb_table):
    B, L, V = probas.shape
    N = B * L
    p2 = probas.reshape(N, V)
    key = jax.random.key(42)
    gum3 = jnp.stack(
        [jax.random.gumbel(jax.random.fold_in(key, it), (N, _TOP_K),
                           jnp.float32) for it in range(_N_ITER)], axis=0)
    x2, samples = _pass1(p2, gum3)

    embs = _sc_gather(samples.reshape(-1), batch_vocab, emb_table)
    embs3 = embs.reshape(B, L, _N_ITER, 128).transpose(2, 0, 1, 3)
    ms = _det(embs3.reshape(_N_ITER * B, L, 128), h_d)
    return (x2.reshape(B, L, V), ms.reshape(B))


# speedup vs baseline: 77.2866x; 1.4322x over previous
import functools

import jax
import jax.numpy as jnp
from jax import lax
from jax.experimental import pallas as pl
from jax.experimental.pallas import tpu as pltpu
from jax.experimental.pallas import tpu_sc as plsc

_TOP_K = 16
_N_ITER = 4
_EARLY = 2
_W = 512
_R = 16
_L = 32


def _pass1_body(p_ref, g_ref, x_ref, smp_ref, win_ref):
    R, V = p_ref.shape
    K = _TOP_K
    W = _W
    nfull = V // W
    tailw = V - nfull * W
    C = nfull + (1 if tailw else 0)
    CP = 256

    lane_cp = lax.broadcasted_iota(jnp.int32, (R, CP), 1)
    M1 = jnp.full((R, CP), -2.0, dtype=jnp.float32)
    rs = jnp.zeros((R, 1), dtype=jnp.float32)
    for c in range(nfull):
        ch = p_ref[:, c * W:(c + 1) * W]
        rs = rs + jnp.sum(ch, axis=1, keepdims=True)
        mx = jnp.max(ch, axis=1, keepdims=True)
        M1 = jnp.where(lane_cp == c, mx, M1)
    tl_all = p_ref[:, nfull * W:]
    rs = rs + jnp.sum(tl_all, axis=1, keepdims=True)
    mx = jnp.max(tl_all, axis=1, keepdims=True)
    M1 = jnp.where(lane_cp == nfull, mx, M1)

    recip = 0.5 / (0.5 * rs)
    for c in range(nfull):
        sl = slice(c * W, (c + 1) * W)
        x_ref[:, sl] = p_ref[:, sl] * recip
    x_ref[:, nfull * W:] = tl_all * recip

    BIG = jnp.int32(1 << 30)
    lane_w = lax.broadcasted_iota(jnp.int32, (R, W), 1)
    lane_t = lax.broadcasted_iota(jnp.int32, (R, tailw), 1)
    k_iota = lax.broadcasted_iota(jnp.int32, (R, K), 1)
    it_iota = lax.broadcasted_iota(jnp.int32, (R, _N_ITER), 1)
    tbase = nfull * W
    glpos_t = lane_t + tbase
    vals = jnp.zeros((R, K), dtype=jnp.float32)
    idxs = jnp.zeros((R, K), dtype=jnp.int32)
    for k in range(K):
        m8 = jnp.max(M1, axis=1, keepdims=True)
        c8 = jnp.min(jnp.where(M1 == m8, lane_cp, BIG), axis=1,
                     keepdims=True)
        cw8 = jnp.minimum(c8, nfull - 1)
        for r in range(R):
            start = pl.multiple_of(cw8[r, 0] * W, W)
            win_ref[pl.ds(r, 1), :] = p_ref[pl.ds(r, 1), pl.ds(start, W)]
        g = win_ref[...]
        glpos = lane_w + cw8 * W
        is_tail = c8 == (C - 1)
        p8w = jnp.min(jnp.where(g == m8, glpos, BIG), axis=1, keepdims=True)
        p8t = jnp.min(jnp.where(tl_all == m8, glpos_t, BIG), axis=1,
                      keepdims=True)
        p8 = jnp.where(is_tail, p8t, p8w)
        vals = jnp.where(k_iota == k, m8, vals)
        idxs = jnp.where(k_iota == k, p8, idxs)
        if k < K - 1:
            keepw = jnp.logical_or(g < m8,
                                   jnp.logical_and(g == m8, glpos > p8))
            nmw = jnp.max(jnp.where(keepw, g, -2.0), axis=1, keepdims=True)
            keept = jnp.logical_or(tl_all < m8,
                                   jnp.logical_and(tl_all == m8,
                                                   glpos_t > p8))
            nmt = jnp.max(jnp.where(keept, tl_all, -2.0), axis=1,
                          keepdims=True)
            newm = jnp.where(is_tail, nmt, nmw)
            M1 = jnp.where(lane_cp == c8, newm, M1)

    lgr = jnp.log(vals)
    rowid = pl.program_id(0) * R + lax.broadcasted_iota(
        jnp.int32, (R, 1), 0)
    is_last = rowid % _L == (_L - 1)
    acc = jnp.zeros((R, _N_ITER), dtype=jnp.int32)
    for it in range(_N_ITER):
        s = lgr + g_ref[it]
        m2 = jnp.max(s, axis=1, keepdims=True)
        ch = jnp.min(jnp.where(s == m2, k_iota, BIG), axis=1, keepdims=True)
        ch = jnp.where(is_last, 0, ch)
        samp = jnp.min(jnp.where(k_iota == ch, idxs, BIG), axis=1,
                       keepdims=True)
        acc = jnp.where(it_iota == it, samp, acc)
    smp_ref[...] = acc


def _pass1(p2, gum3):
    N, V = p2.shape
    grid = N // _R
    return pl.pallas_call(
        _pass1_body,
        grid=(grid,),
        in_specs=[
            pl.BlockSpec((_R, V), lambda i: (i, 0)),
            pl.BlockSpec((_N_ITER, _R, _TOP_K), lambda i: (0, i, 0)),
        ],
        out_specs=[
            pl.BlockSpec((_R, V), lambda i: (i, 0)),
            pl.BlockSpec((_R, _N_ITER), lambda i: (i, 0)),
        ],
        out_shape=[
            jax.ShapeDtypeStruct((N, V), jnp.float32),
            jax.ShapeDtypeStruct((N, _N_ITER), jnp.int32),
        ],
        scratch_shapes=[pltpu.VMEM((_R, _W), jnp.float32)],
    )(p2, gum3)


def _sc_gather(samples1, batch_vocab, emb_table):
    n_tasks = samples1.shape[0]
    D = emb_table.shape[1]
    info = plsc.get_sparse_core_info()
    NC, NS = info.num_cores, info.num_subcores
    NW = NC * NS
    per_w = n_tasks // NW
    mesh = plsc.VectorSubcoreMesh(core_axis_name="c", subcore_axis_name="s")

    @functools.partial(
        pl.kernel,
        mesh=mesh,
        out_type=jax.ShapeDtypeStruct((n_tasks, D), jnp.float32),
        scratch_types=[
            pltpu.VMEM((per_w,), jnp.int32),
            pltpu.VMEM((per_w,), jnp.int32),
            pltpu.VMEM((per_w, D), jnp.float32),
            pltpu.SemaphoreType.DMA,
        ],
    )
    def body(s_hbm, bv_hbm, emb_hbm, out_hbm, smp_v, wi_v, rows_v, sem):
        wid = lax.axis_index("s") * NC + lax.axis_index("c")
        base = wid * per_w
        pltpu.sync_copy(s_hbm.at[pl.ds(base, per_w)], smp_v)
        pltpu.async_copy(bv_hbm.at[smp_v], wi_v, sem).wait()
        pltpu.async_copy(emb_hbm.at[wi_v], rows_v, sem).wait()
        pltpu.sync_copy(rows_v, out_hbm.at[pl.ds(base, per_w)])

    return body(samples1, batch_vocab, emb_table)


def _det_body(e_ref, h_ref, out_ref):
    B = h_ref.shape[0]
    Lm = h_ref.shape[1]
    NM = _N_ITER * B
    row_i = lax.broadcasted_iota(jnp.int32, (Lm, Lm), 0)
    col_i = lax.broadcasted_iota(jnp.int32, (Lm, Lm), 1)
    b_iota = lax.broadcasted_iota(jnp.int32, (1, B), 1)
    dets = [None] * NM
    for m in range(NM):
        b = m % B
        it = m // B
        G = (e_ref[b, :, it, :] + h_ref[b]) / 10.0
        A0 = lax.dot_general(G, G, (((1,), (1,)), ((), ())),
                             preferred_element_type=jnp.float32)

        def step(j, carry):
            A, d = carry
            rsel = row_i == j
            csel = col_i == j
            prow = jnp.sum(jnp.where(rsel, A, 0.0), axis=0, keepdims=True)
            pv = jnp.sum(jnp.where(jnp.logical_and(rsel, csel), A, 0.0))
            fcol = jnp.sum(jnp.where(csel, A, 0.0), axis=1, keepdims=True) / pv
            upd_mask = row_i > j
            A = A - jnp.where(upd_mask, fcol * prow, 0.0)
            return A, d * pv

        _, det_m = lax.fori_loop(0, Lm, step,
                                 (A0, jnp.float32(1.0)), unroll=False)
        dets[m] = det_m
    S = [dets[i * B:(i + 1) * B] for i in range(_N_ITER)]

    def rowvec(ds):
        v = jnp.zeros((1, B), jnp.float32)
        for b in range(B):
            v = jnp.where(b_iota == b, ds[b], v)
        return v

    Srows = [rowvec(ds) for ds in S]
    ms = Srows[0]
    cnt = jnp.int32(0)
    done = jnp.bool_(False)
    for it in range(1, _N_ITER):
        s = Srows[it]
        mvec = ms < s
        any_m = jnp.any(mvec)
        cnt_new = jnp.where(any_m, jnp.int32(0), cnt + 1)
        cnt = jnp.where(done, cnt, cnt_new)
        hit = cnt >= _EARLY
        upd = jnp.logical_and(jnp.logical_not(done), jnp.logical_not(hit))
        done = jnp.logical_or(done, hit)
        ms = jnp.where(jnp.logical_and(upd, mvec), s, ms)
    out_ref[...] = ms


def _det(embs3, h_d):
    B = h_d.shape[0]
    return pl.pallas_call(
        _det_body,
        out_shape=jax.ShapeDtypeStruct((1, B), jnp.float32),
    )(embs3, h_d)


def kernel(probas, h_d, mask, batch_vocab, emb_table):
    B, L, V = probas.shape
    N = B * L
    p2 = probas.reshape(N, V)
    key = jax.random.key(42)
    gum3 = jnp.stack(
        [jax.random.gumbel(jax.random.fold_in(key, it), (N, _TOP_K),
                           jnp.float32) for it in range(_N_ITER)], axis=0)
    x2, samples = _pass1(p2, gum3)

    embs = _sc_gather(samples.reshape(-1), batch_vocab, emb_table)
    ms = _det(embs.reshape(B, L, _N_ITER, 128), h_d)
    return (x2.reshape(B, L, V), ms.reshape(B))


# speedup vs baseline: 134.6881x; 1.7427x over previous
import functools

import jax
import jax.numpy as jnp
from jax import lax
from jax.experimental import pallas as pl
from jax.experimental.pallas import tpu as pltpu
from jax.experimental.pallas import tpu_sc as plsc

_TOP_K = 16
_N_ITER = 4
_EARLY = 2
_W = 512
_R = 16
_L = 32


def _pass1_body(p_ref, g_ref, x_ref, smp_ref, win_ref):
    R, V = p_ref.shape
    K = _TOP_K
    W = _W
    nfull = V // W
    tailw = V - nfull * W
    C = nfull + (1 if tailw else 0)
    CP = 256

    lane_cp = lax.broadcasted_iota(jnp.int32, (R, CP), 1)
    M1 = jnp.full((R, CP), -2.0, dtype=jnp.float32)
    rs = jnp.zeros((R, 1), dtype=jnp.float32)
    for c in range(nfull):
        ch = p_ref[:, c * W:(c + 1) * W]
        rs = rs + jnp.sum(ch, axis=1, keepdims=True)
        mx = jnp.max(ch, axis=1, keepdims=True)
        M1 = jnp.where(lane_cp == c, mx, M1)
    tl_all = p_ref[:, nfull * W:]
    rs = rs + jnp.sum(tl_all, axis=1, keepdims=True)
    mx = jnp.max(tl_all, axis=1, keepdims=True)
    M1 = jnp.where(lane_cp == nfull, mx, M1)

    recip = 0.5 / (0.5 * rs)
    for c in range(nfull):
        sl = slice(c * W, (c + 1) * W)
        x_ref[:, sl] = p_ref[:, sl] * recip
    x_ref[:, nfull * W:] = tl_all * recip

    BIG = jnp.int32(1 << 30)
    lane_w = lax.broadcasted_iota(jnp.int32, (R, W), 1)
    lane_t = lax.broadcasted_iota(jnp.int32, (R, tailw), 1)
    k_iota = lax.broadcasted_iota(jnp.int32, (R, K), 1)
    it_iota = lax.broadcasted_iota(jnp.int32, (R, _N_ITER), 1)
    tbase = nfull * W
    glpos_t = lane_t + tbase
    vals = jnp.zeros((R, K), dtype=jnp.float32)
    idxs = jnp.zeros((R, K), dtype=jnp.int32)
    for k in range(K):
        m8 = jnp.max(M1, axis=1, keepdims=True)
        c8 = jnp.min(jnp.where(M1 == m8, lane_cp, BIG), axis=1,
                     keepdims=True)
        cw8 = jnp.minimum(c8, nfull - 1)
        for r in range(R):
            start = pl.multiple_of(cw8[r, 0] * W, W)
            win_ref[pl.ds(r, 1), :] = p_ref[pl.ds(r, 1), pl.ds(start, W)]
        g = win_ref[...]
        glpos = lane_w + cw8 * W
        is_tail = c8 == (C - 1)
        p8w = jnp.min(jnp.where(g == m8, glpos, BIG), axis=1, keepdims=True)
        p8t = jnp.min(jnp.where(tl_all == m8, glpos_t, BIG), axis=1,
                      keepdims=True)
        p8 = jnp.where(is_tail, p8t, p8w)
        vals = jnp.where(k_iota == k, m8, vals)
        idxs = jnp.where(k_iota == k, p8, idxs)
        if k < K - 1:
            keepw = jnp.logical_or(g < m8,
                                   jnp.logical_and(g == m8, glpos > p8))
            nmw = jnp.max(jnp.where(keepw, g, -2.0), axis=1, keepdims=True)
            keept = jnp.logical_or(tl_all < m8,
                                   jnp.logical_and(tl_all == m8,
                                                   glpos_t > p8))
            nmt = jnp.max(jnp.where(keept, tl_all, -2.0), axis=1,
                          keepdims=True)
            newm = jnp.where(is_tail, nmt, nmw)
            M1 = jnp.where(lane_cp == c8, newm, M1)

    lgr = jnp.log(vals)
    rowid = pl.program_id(0) * R + lax.broadcasted_iota(
        jnp.int32, (R, 1), 0)
    is_last = rowid % _L == (_L - 1)
    acc = jnp.zeros((R, _N_ITER), dtype=jnp.int32)
    for it in range(_N_ITER):
        s = lgr + g_ref[it]
        m2 = jnp.max(s, axis=1, keepdims=True)
        ch = jnp.min(jnp.where(s == m2, k_iota, BIG), axis=1, keepdims=True)
        ch = jnp.where(is_last, 0, ch)
        samp = jnp.min(jnp.where(k_iota == ch, idxs, BIG), axis=1,
                       keepdims=True)
        acc = jnp.where(it_iota == it, samp, acc)
    smp_ref[...] = acc


def _pass1(p2, gum3):
    N, V = p2.shape
    grid = N // _R
    return pl.pallas_call(
        _pass1_body,
        grid=(grid,),
        in_specs=[
            pl.BlockSpec((_R, V), lambda i: (i, 0)),
            pl.BlockSpec((_N_ITER, _R, _TOP_K), lambda i: (0, i, 0)),
        ],
        out_specs=[
            pl.BlockSpec((_R, V), lambda i: (i, 0)),
            pl.BlockSpec((_R, _N_ITER), lambda i: (i, 0)),
        ],
        out_shape=[
            jax.ShapeDtypeStruct((N, V), jnp.float32),
            jax.ShapeDtypeStruct((N, _N_ITER), jnp.int32),
        ],
        scratch_shapes=[pltpu.VMEM((_R, _W), jnp.float32)],
    )(p2, gum3)


def _sc_gather(samples1, batch_vocab, emb_table):
    n_tasks = samples1.shape[0]
    D = emb_table.shape[1]
    info = plsc.get_sparse_core_info()
    NC, NS = info.num_cores, info.num_subcores
    NW = NC * NS
    per_w = n_tasks // NW
    mesh = plsc.VectorSubcoreMesh(core_axis_name="c", subcore_axis_name="s")

    @functools.partial(
        pl.kernel,
        mesh=mesh,
        out_type=jax.ShapeDtypeStruct((n_tasks, D), jnp.float32),
        scratch_types=[
            pltpu.VMEM((per_w,), jnp.int32),
            pltpu.VMEM((per_w,), jnp.int32),
            pltpu.VMEM((per_w, D), jnp.float32),
            pltpu.SemaphoreType.DMA,
        ],
    )
    def body(s_hbm, bv_hbm, emb_hbm, out_hbm, smp_v, wi_v, rows_v, sem):
        wid = lax.axis_index("s") * NC + lax.axis_index("c")
        base = wid * per_w
        pltpu.sync_copy(s_hbm.at[pl.ds(base, per_w)], smp_v)
        pltpu.async_copy(bv_hbm.at[smp_v], wi_v, sem).wait()
        pltpu.async_copy(emb_hbm.at[wi_v], rows_v, sem).wait()
        pltpu.sync_copy(rows_v, out_hbm.at[pl.ds(base, per_w)])

    return body(samples1, batch_vocab, emb_table)


def _det_body(e_ref, h_ref, out_ref):
    B = h_ref.shape[0]
    Lm = h_ref.shape[1]
    NM = _N_ITER * B
    row_i3 = lax.broadcasted_iota(jnp.int32, (1, Lm, Lm), 1)
    mats = []
    for m in range(NM):
        b = m % B
        it = m // B
        G = (e_ref[b, :, it, :] + h_ref[b]) / 10.0
        A0 = lax.dot_general(G, G, (((1,), (1,)), ((), ())),
                             preferred_element_type=jnp.float32)
        mats.append(A0[None])
    A = jnp.concatenate(mats, axis=0)
    d = jnp.ones((NM, 1, 1), jnp.float32)
    for j in range(Lm):
        prow = A[:, j:j + 1, :]
        pv = A[:, j:j + 1, j:j + 1]
        fcol = A[:, :, j:j + 1] / pv
        A = A - jnp.where(row_i3 > j, fcol * prow, 0.0)
        d = d * pv
    dets_row = jnp.reshape(d, (1, NM))
    Srows = [dets_row[:, i * B:(i + 1) * B] for i in range(_N_ITER)]
    ms = Srows[0]
    cnt = jnp.int32(0)
    done = jnp.bool_(False)
    for it in range(1, _N_ITER):
        s = Srows[it]
        mvec = ms < s
        any_m = jnp.any(mvec)
        cnt_new = jnp.where(any_m, jnp.int32(0), cnt + 1)
        cnt = jnp.where(done, cnt, cnt_new)
        hit = cnt >= _EARLY
        upd = jnp.logical_and(jnp.logical_not(done), jnp.logical_not(hit))
        done = jnp.logical_or(done, hit)
        ms = jnp.where(jnp.logical_and(upd, mvec), s, ms)
    out_ref[...] = ms


def _det(embs3, h_d):
    B = h_d.shape[0]
    return pl.pallas_call(
        _det_body,
        out_shape=jax.ShapeDtypeStruct((1, B), jnp.float32),
    )(embs3, h_d)


def kernel(probas, h_d, mask, batch_vocab, emb_table):
    B, L, V = probas.shape
    N = B * L
    p2 = probas.reshape(N, V)
    key = jax.random.key(42)
    gum3 = jnp.stack(
        [jax.random.gumbel(jax.random.fold_in(key, it), (N, _TOP_K),
                           jnp.float32) for it in range(_N_ITER)], axis=0)
    x2, samples = _pass1(p2, gum3)

    embs = _sc_gather(samples.reshape(-1), batch_vocab, emb_table)
    ms = _det(embs.reshape(B, L, _N_ITER, 128), h_d)
    return (x2.reshape(B, L, V), ms.reshape(B))


# speedup vs baseline: 225.3018x; 1.6728x over previous
import functools

import jax
import jax.numpy as jnp
from jax import lax
from jax.experimental import pallas as pl
from jax.experimental.pallas import tpu as pltpu
from jax.experimental.pallas import tpu_sc as plsc

_TOP_K = 16
_N_ITER = 4
_EARLY = 2
_W = 512
_R = 32
_L = 32


def _pass1_body(p_ref, g_ref, x_ref, smp_ref, win_ref):
    R, V = p_ref.shape
    K = _TOP_K
    W = _W
    nfull = V // W
    tailw = V - nfull * W
    C = nfull + (1 if tailw else 0)
    CP = 256

    lane_cp = lax.broadcasted_iota(jnp.int32, (R, CP), 1)
    M1 = jnp.full((R, CP), -2.0, dtype=jnp.float32)
    rs = jnp.zeros((R, 1), dtype=jnp.float32)
    for c in range(nfull):
        ch = p_ref[:, c * W:(c + 1) * W]
        rs = rs + jnp.sum(ch, axis=1, keepdims=True)
        mx = jnp.max(ch, axis=1, keepdims=True)
        M1 = jnp.where(lane_cp == c, mx, M1)
    tl_all = p_ref[:, nfull * W:]
    rs = rs + jnp.sum(tl_all, axis=1, keepdims=True)
    mx = jnp.max(tl_all, axis=1, keepdims=True)
    M1 = jnp.where(lane_cp == nfull, mx, M1)

    recip = 0.5 / (0.5 * rs)
    x_ref[:, nfull * W:] = tl_all * recip
    xw_per_k = -(-nfull // _TOP_K)

    BIG = jnp.int32(1 << 30)
    lane_w = lax.broadcasted_iota(jnp.int32, (R, W), 1)
    lane_t = lax.broadcasted_iota(jnp.int32, (R, tailw), 1)
    k_iota = lax.broadcasted_iota(jnp.int32, (R, K), 1)
    it_iota = lax.broadcasted_iota(jnp.int32, (R, _N_ITER), 1)
    tbase = nfull * W
    glpos_t = lane_t + tbase
    vals = jnp.zeros((R, K), dtype=jnp.float32)
    idxs = jnp.zeros((R, K), dtype=jnp.int32)
    for k in range(K):
        m8 = jnp.max(M1, axis=1, keepdims=True)
        c8 = jnp.min(jnp.where(M1 == m8, lane_cp, BIG), axis=1,
                     keepdims=True)
        cw8 = jnp.minimum(c8, nfull - 1)
        for r in range(R):
            start = pl.multiple_of(cw8[r, 0] * W, W)
            win_ref[pl.ds(r, 1), :] = p_ref[pl.ds(r, 1), pl.ds(start, W)]
        g = win_ref[...]
        glpos = lane_w + cw8 * W
        is_tail = c8 == (C - 1)
        p8w = jnp.min(jnp.where(g == m8, glpos, BIG), axis=1, keepdims=True)
        p8t = jnp.min(jnp.where(tl_all == m8, glpos_t, BIG), axis=1,
                      keepdims=True)
        p8 = jnp.where(is_tail, p8t, p8w)
        vals = jnp.where(k_iota == k, m8, vals)
        idxs = jnp.where(k_iota == k, p8, idxs)
        if k < K - 1:
            keepw = jnp.logical_or(g < m8,
                                   jnp.logical_and(g == m8, glpos > p8))
            nmw = jnp.max(jnp.where(keepw, g, -2.0), axis=1, keepdims=True)
            keept = jnp.logical_or(tl_all < m8,
                                   jnp.logical_and(tl_all == m8,
                                                   glpos_t > p8))
            nmt = jnp.max(jnp.where(keept, tl_all, -2.0), axis=1,
                          keepdims=True)
            newm = jnp.where(is_tail, nmt, nmw)
            M1 = jnp.where(lane_cp == c8, newm, M1)
        for c in range(k * xw_per_k, min((k + 1) * xw_per_k, nfull)):
            sl = slice(c * W, (c + 1) * W)
            x_ref[:, sl] = p_ref[:, sl] * recip

    lgr = jnp.log(vals)
    rowid = pl.program_id(0) * R + lax.broadcasted_iota(
        jnp.int32, (R, 1), 0)
    is_last = rowid % _L == (_L - 1)
    acc = jnp.zeros((R, _N_ITER), dtype=jnp.int32)
    for it in range(_N_ITER):
        s = lgr + g_ref[it]
        m2 = jnp.max(s, axis=1, keepdims=True)
        ch = jnp.min(jnp.where(s == m2, k_iota, BIG), axis=1, keepdims=True)
        ch = jnp.where(is_last, 0, ch)
        samp = jnp.min(jnp.where(k_iota == ch, idxs, BIG), axis=1,
                       keepdims=True)
        acc = jnp.where(it_iota == it, samp, acc)
    smp_ref[...] = acc


def _pass1(p2, gum3):
    N, V = p2.shape
    grid = N // _R
    return pl.pallas_call(
        _pass1_body,
        grid=(grid,),
        in_specs=[
            pl.BlockSpec((_R, V), lambda i: (i, 0)),
            pl.BlockSpec((_N_ITER, _R, _TOP_K), lambda i: (0, i, 0)),
        ],
        out_specs=[
            pl.BlockSpec((_R, V), lambda i: (i, 0)),
            pl.BlockSpec((_R, _N_ITER), lambda i: (i, 0)),
        ],
        out_shape=[
            jax.ShapeDtypeStruct((N, V), jnp.float32),
            jax.ShapeDtypeStruct((N, _N_ITER), jnp.int32),
        ],
        scratch_shapes=[pltpu.VMEM((_R, _W), jnp.float32)],
    )(p2, gum3)


def _sc_gather(samples1, batch_vocab, emb_table):
    n_tasks = samples1.shape[0]
    D = emb_table.shape[1]
    info = plsc.get_sparse_core_info()
    NC, NS = info.num_cores, info.num_subcores
    NW = NC * NS
    per_w = n_tasks // NW
    mesh = plsc.VectorSubcoreMesh(core_axis_name="c", subcore_axis_name="s")

    @functools.partial(
        pl.kernel,
        mesh=mesh,
        out_type=jax.ShapeDtypeStruct((n_tasks, D), jnp.float32),
        scratch_types=[
            pltpu.VMEM((per_w,), jnp.int32),
            pltpu.VMEM((per_w,), jnp.int32),
            pltpu.VMEM((per_w, D), jnp.float32),
            pltpu.SemaphoreType.DMA,
        ],
    )
    def body(s_hbm, bv_hbm, emb_hbm, out_hbm, smp_v, wi_v, rows_v, sem):
        wid = lax.axis_index("s") * NC + lax.axis_index("c")
        base = wid * per_w
        pltpu.sync_copy(s_hbm.at[pl.ds(base, per_w)], smp_v)
        pltpu.async_copy(bv_hbm.at[smp_v], wi_v, sem).wait()
        pltpu.async_copy(emb_hbm.at[wi_v], rows_v, sem).wait()
        pltpu.sync_copy(rows_v, out_hbm.at[pl.ds(base, per_w)])

    return body(samples1, batch_vocab, emb_table)


def _det_body(e_ref, h_ref, out_ref):
    B = h_ref.shape[0]
    Lm = h_ref.shape[1]
    NM = _N_ITER * B
    row_i3 = lax.broadcasted_iota(jnp.int32, (1, Lm, Lm), 1)
    mats = []
    for m in range(NM):
        b = m % B
        it = m // B
        G = (e_ref[b, :, it, :] + h_ref[b]) / 10.0
        A0 = lax.dot_general(G, G, (((1,), (1,)), ((), ())),
                             preferred_element_type=jnp.float32)
        mats.append(A0[None])
    A = jnp.concatenate(mats, axis=0)
    d = jnp.ones((NM, 1, 1), jnp.float32)
    for j in range(Lm):
        prow = A[:, j:j + 1, :]
        pv = A[:, j:j + 1, j:j + 1]
        fcol = A[:, :, j:j + 1] / pv
        A = A - jnp.where(row_i3 > j, fcol * prow, 0.0)
        d = d * pv
    dets_row = jnp.reshape(d, (1, NM))
    Srows = [dets_row[:, i * B:(i + 1) * B] for i in range(_N_ITER)]
    ms = Srows[0]
    cnt = jnp.int32(0)
    done = jnp.bool_(False)
    for it in range(1, _N_ITER):
        s = Srows[it]
        mvec = ms < s
        any_m = jnp.any(mvec)
        cnt_new = jnp.where(any_m, jnp.int32(0), cnt + 1)
        cnt = jnp.where(done, cnt, cnt_new)
        hit = cnt >= _EARLY
        upd = jnp.logical_and(jnp.logical_not(done), jnp.logical_not(hit))
        done = jnp.logical_or(done, hit)
        ms = jnp.where(jnp.logical_and(upd, mvec), s, ms)
    out_ref[...] = ms


def _det(embs3, h_d):
    B = h_d.shape[0]
    return pl.pallas_call(
        _det_body,
        out_shape=jax.ShapeDtypeStruct((1, B), jnp.float32),
    )(embs3, h_d)


def kernel(probas, h_d, mask, batch_vocab, emb_table):
    B, L, V = probas.shape
    N = B * L
    p2 = probas.reshape(N, V)
    key = jax.random.key(42)
    keys = jax.vmap(lambda it: jax.random.fold_in(key, it))(
        jnp.arange(_N_ITER, dtype=jnp.uint32))
    gum3 = jax.vmap(
        lambda k: jax.random.gumbel(k, (N, _TOP_K), jnp.float32))(keys)
    x2, samples = _pass1(p2, gum3)

    embs = _sc_gather(samples.reshape(-1), batch_vocab, emb_table)
    ms = _det(embs.reshape(B, L, _N_ITER, 128), h_d)
    return (x2.reshape(B, L, V), ms.reshape(B))


# speedup vs baseline: 225.4521x; 1.0007x over previous
import functools

import jax
import jax.numpy as jnp
from jax import lax
from jax.experimental import pallas as pl
from jax.experimental.pallas import tpu as pltpu
from jax.experimental.pallas import tpu_sc as plsc

_TOP_K = 16
_N_ITER = 4
_EARLY = 2
_W = 512
_R = 32
_L = 32


def _pass1_body(p_ref, g_ref, x_ref, smp_ref, win_ref):
    R, V = p_ref.shape
    K = _TOP_K
    W = _W
    nfull = V // W
    tailw = V - nfull * W
    C = nfull + (1 if tailw else 0)
    CP = 256

    lane_cp = lax.broadcasted_iota(jnp.int32, (R, CP), 1)
    M1 = jnp.full((R, CP), -2.0, dtype=jnp.float32)
    rs = jnp.zeros((R, 1), dtype=jnp.float32)
    for c in range(nfull):
        ch = p_ref[:, c * W:(c + 1) * W]
        rs = rs + jnp.sum(ch, axis=1, keepdims=True)
        mx = jnp.max(ch, axis=1, keepdims=True)
        M1 = jnp.where(lane_cp == c, mx, M1)
    tl_all = p_ref[:, nfull * W:]
    rs = rs + jnp.sum(tl_all, axis=1, keepdims=True)
    mx = jnp.max(tl_all, axis=1, keepdims=True)
    M1 = jnp.where(lane_cp == nfull, mx, M1)

    recip = 0.5 / (0.5 * rs)
    x_ref[:, nfull * W:] = tl_all * recip
    xw_per_k = -(-nfull // _TOP_K)

    BIG = jnp.int32(1 << 30)
    lane_w = lax.broadcasted_iota(jnp.int32, (R, W), 1)
    lane_t = lax.broadcasted_iota(jnp.int32, (R, tailw), 1)
    k_iota = lax.broadcasted_iota(jnp.int32, (R, K), 1)
    it_iota = lax.broadcasted_iota(jnp.int32, (R, _N_ITER), 1)
    tbase = nfull * W
    glpos_t = lane_t + tbase
    vals = jnp.zeros((R, K), dtype=jnp.float32)
    idxs = jnp.zeros((R, K), dtype=jnp.int32)
    for k in range(K):
        m8 = jnp.max(M1, axis=1, keepdims=True)
        c8 = jnp.min(jnp.where(M1 == m8, lane_cp, BIG), axis=1,
                     keepdims=True)
        cw8 = jnp.minimum(c8, nfull - 1)
        for r in range(R):
            start = pl.multiple_of(cw8[r, 0] * W, W)
            win_ref[pl.ds(r, 1), :] = p_ref[pl.ds(r, 1), pl.ds(start, W)]
        g = win_ref[...]
        glpos = lane_w + cw8 * W
        is_tail = c8 == (C - 1)
        p8w = jnp.min(jnp.where(g == m8, glpos, BIG), axis=1, keepdims=True)
        p8t = jnp.min(jnp.where(tl_all == m8, glpos_t, BIG), axis=1,
                      keepdims=True)
        p8 = jnp.where(is_tail, p8t, p8w)
        vals = jnp.where(k_iota == k, m8, vals)
        idxs = jnp.where(k_iota == k, p8, idxs)
        if k < K - 1:
            keepw = jnp.logical_or(g < m8,
                                   jnp.logical_and(g == m8, glpos > p8))
            nmw = jnp.max(jnp.where(keepw, g, -2.0), axis=1, keepdims=True)
            keept = jnp.logical_or(tl_all < m8,
                                   jnp.logical_and(tl_all == m8,
                                                   glpos_t > p8))
            nmt = jnp.max(jnp.where(keept, tl_all, -2.0), axis=1,
                          keepdims=True)
            newm = jnp.where(is_tail, nmt, nmw)
            M1 = jnp.where(lane_cp == c8, newm, M1)
        for c in range(k * xw_per_k, min((k + 1) * xw_per_k, nfull)):
            sl = slice(c * W, (c + 1) * W)
            x_ref[:, sl] = p_ref[:, sl] * recip

    lgr = jnp.log(vals)
    rowid = pl.program_id(0) * R + lax.broadcasted_iota(
        jnp.int32, (R, 1), 0)
    is_last = rowid % _L == (_L - 1)
    acc = jnp.zeros((R, _N_ITER), dtype=jnp.int32)
    for it in range(_N_ITER):
        s = lgr + g_ref[it]
        m2 = jnp.max(s, axis=1, keepdims=True)
        ch = jnp.min(jnp.where(s == m2, k_iota, BIG), axis=1, keepdims=True)
        ch = jnp.where(is_last, 0, ch)
        samp = jnp.min(jnp.where(k_iota == ch, idxs, BIG), axis=1,
                       keepdims=True)
        acc = jnp.where(it_iota == it, samp, acc)
    smp_ref[...] = acc


@functools.lru_cache(maxsize=2)
def _gumbel_const(N):
    key = jax.random.key(42)
    keys = jax.vmap(lambda it: jax.random.fold_in(key, it))(
        jnp.arange(_N_ITER, dtype=jnp.uint32))
    return jax.vmap(
        lambda k: jax.random.gumbel(k, (N, _TOP_K), jnp.float32))(keys)


def _pass1(p2, gum3):
    N, V = p2.shape
    grid = N // _R
    return pl.pallas_call(
        _pass1_body,
        grid=(grid,),
        in_specs=[
            pl.BlockSpec((_R, V), lambda i: (i, 0)),
            pl.BlockSpec((_N_ITER, _R, _TOP_K), lambda i: (0, i, 0)),
        ],
        out_specs=[
            pl.BlockSpec((_R, V), lambda i: (i, 0)),
            pl.BlockSpec((_R, _N_ITER), lambda i: (i, 0)),
        ],
        out_shape=[
            jax.ShapeDtypeStruct((N, V), jnp.float32),
            jax.ShapeDtypeStruct((N, _N_ITER), jnp.int32),
        ],
        scratch_shapes=[pltpu.VMEM((_R, _W), jnp.float32)],
    )(p2, gum3)


def _sc_gather(samples1, batch_vocab, emb_table):
    n_tasks = samples1.shape[0]
    D = emb_table.shape[1]
    info = plsc.get_sparse_core_info()
    NC, NS = info.num_cores, info.num_subcores
    NW = NC * NS
    per_w = n_tasks // NW
    mesh = plsc.VectorSubcoreMesh(core_axis_name="c", subcore_axis_name="s")

    @functools.partial(
        pl.kernel,
        mesh=mesh,
        out_type=jax.ShapeDtypeStruct((n_tasks, D), jnp.float32),
        scratch_types=[
            pltpu.VMEM((per_w,), jnp.int32),
            pltpu.VMEM((per_w,), jnp.int32),
            pltpu.VMEM((per_w, D), jnp.float32),
            pltpu.SemaphoreType.DMA,
        ],
    )
    def body(s_hbm, bv_hbm, emb_hbm, out_hbm, smp_v, wi_v, rows_v, sem):
        wid = lax.axis_index("s") * NC + lax.axis_index("c")
        base = wid * per_w
        pltpu.sync_copy(s_hbm.at[pl.ds(base, per_w)], smp_v)
        pltpu.async_copy(bv_hbm.at[smp_v], wi_v, sem).wait()
        pltpu.async_copy(emb_hbm.at[wi_v], rows_v, sem).wait()
        pltpu.sync_copy(rows_v, out_hbm.at[pl.ds(base, per_w)])

    return body(samples1, batch_vocab, emb_table)


def _det_body(e_ref, h_ref, out_ref):
    B = h_ref.shape[0]
    Lm = h_ref.shape[1]
    NM = _N_ITER * B
    row_i3 = lax.broadcasted_iota(jnp.int32, (1, Lm, Lm), 1)
    mats = []
    for m in range(NM):
        b = m % B
        it = m // B
        G = (e_ref[b, :, it, :] + h_ref[b]) / 10.0
        A0 = lax.dot_general(G, G, (((1,), (1,)), ((), ())),
                             preferred_element_type=jnp.float32)
        mats.append(A0[None])
    A = jnp.concatenate(mats, axis=0)
    d = jnp.ones((NM, 1, 1), jnp.float32)
    for j in range(Lm):
        prow = A[:, j:j + 1, :]
        pv = A[:, j:j + 1, j:j + 1]
        fcol = A[:, :, j:j + 1] / pv
        A = A - jnp.where(row_i3 > j, fcol * prow, 0.0)
        d = d * pv
    dets_row = jnp.reshape(d, (1, NM))
    Srows = [dets_row[:, i * B:(i + 1) * B] for i in range(_N_ITER)]
    ms = Srows[0]
    cnt = jnp.int32(0)
    done = jnp.bool_(False)
    for it in range(1, _N_ITER):
        s = Srows[it]
        mvec = ms < s
        any_m = jnp.any(mvec)
        cnt_new = jnp.where(any_m, jnp.int32(0), cnt + 1)
        cnt = jnp.where(done, cnt, cnt_new)
        hit = cnt >= _EARLY
        upd = jnp.logical_and(jnp.logical_not(done), jnp.logical_not(hit))
        done = jnp.logical_or(done, hit)
        ms = jnp.where(jnp.logical_and(upd, mvec), s, ms)
    out_ref[...] = ms


def _det(embs3, h_d):
    B = h_d.shape[0]
    return pl.pallas_call(
        _det_body,
        out_shape=jax.ShapeDtypeStruct((1, B), jnp.float32),
    )(embs3, h_d)


def kernel(probas, h_d, mask, batch_vocab, emb_table):
    B, L, V = probas.shape
    N = B * L
    p2 = probas.reshape(N, V)
    gum3 = _gumbel_const(N)
    x2, samples = _pass1(p2, gum3)

    embs = _sc_gather(samples.reshape(-1), batch_vocab, emb_table)
    ms = _det(embs.reshape(B, L, _N_ITER, 128), h_d)
    return (x2.reshape(B, L, V), ms.reshape(B))
